# Initial kernel scaffold; baseline (speedup 1.0000x reference)
import jax
import jax.numpy as jnp
from jax.experimental import pallas as pl


def kernel(xyz1, xyz2, features1, features2, W1, b1, g1, be1, a1, W2, b2, g2, be2, a2):
    raise NotImplementedError("write your pallas kernel here")


# speedup vs baseline: 25.1128x; 25.1128x over previous
import functools

import jax
import jax.numpy as jnp
from jax.experimental import pallas as pl

_TN = 256


def _knn_conv1_kernel(xyz1_ref, xyz2t_ref, f1_ref, f2_ref, w1a_ref, w1b_ref,
                      b1_ref, y1_ref, stats_ref, *, S):
    x1 = xyz1_ref[0]
    x2t = xyz2t_ref[0]

    dot = jax.lax.dot_general(
        x1.astype(jnp.bfloat16), x2t.astype(jnp.bfloat16),
        dimension_numbers=(((1,), (0,)), ((), ())),
        preferred_element_type=jnp.float32)
    n1 = jnp.sum(x1 * x1, axis=1, keepdims=True)
    n2 = jnp.sum(x2t * x2t, axis=0, keepdims=True)
    d = -2.0 * dot + n1 + n2

    iota = jax.lax.broadcasted_iota(jnp.int32, d.shape, 1)
    big = jnp.float32(jnp.inf)

    idxs = []
    vals = []
    dwork = d
    for _ in range(3):
        m = jnp.min(dwork, axis=1, keepdims=True)
        i = jnp.min(jnp.where(dwork == m, iota, S), axis=1, keepdims=True)
        idxs.append(i)
        vals.append(m)
        dwork = jnp.where(iota == i, big, dwork)

    r1 = 1.0 / (vals[0] + 1e-8)
    r2 = 1.0 / (vals[1] + 1e-8)
    r3 = 1.0 / (vals[2] + 1e-8)
    norm = r1 + r2 + r3
    zero = jnp.float32(0.0)
    wmat = (jnp.where(iota == idxs[0], r1 / norm, zero)
            + jnp.where(iota == idxs[1], r2 / norm, zero)
            + jnp.where(iota == idxs[2], r3 / norm, zero))

    interp = jnp.dot(wmat, f2_ref[0], preferred_element_type=jnp.float32)

    y = (jnp.dot(f1_ref[0], w1a_ref[...], preferred_element_type=jnp.float32)
         + jnp.dot(interp, w1b_ref[...], preferred_element_type=jnp.float32)
         + b1_ref[0:1, :])
    y1_ref[0] = y
    stats_ref[0, 0, 0:1, :] = jnp.sum(y, axis=0, keepdims=True)
    stats_ref[0, 0, 1:2, :] = jnp.sum(y * y, axis=0, keepdims=True)


def _bn_conv2_kernel(y1_ref, ab1_ref, w2_ref, b2_ref, y2_ref, stats_ref):
    alpha = ab1_ref[0:1, :]
    beta = ab1_ref[1:2, :]
    a = ab1_ref[2:3, 0:1]
    z = alpha * y1_ref[0] + beta
    z = jnp.where(z >= 0, z, a * z)
    y = jnp.dot(z, w2_ref[...], preferred_element_type=jnp.float32) \
        + b2_ref[0:1, :]
    y2_ref[0] = y
    stats_ref[0, 0, 0:1, :] = jnp.sum(y, axis=0, keepdims=True)
    stats_ref[0, 0, 1:2, :] = jnp.sum(y * y, axis=0, keepdims=True)


def _bn_out_kernel(y2_ref, ab2_ref, out_ref):
    alpha = ab2_ref[0:1, :]
    beta = ab2_ref[1:2, :]
    a = ab2_ref[2:3, 0:1]
    z = alpha * y2_ref[0] + beta
    out_ref[0] = jnp.where(z >= 0, z, a * z)


def kernel(xyz1, xyz2, features1, features2, W1, b1, g1, be1, a1,
           W2, b2, g2, be2, a2):
    B, N, _ = xyz1.shape
    S = xyz2.shape[1]
    D = features2.shape[-1]
    C1 = W1.shape[0]
    C2 = W2.shape[0]
    TN = _TN
    NT = N // TN
    count = B * N

    xyz2t = jnp.transpose(xyz2, (0, 2, 1))
    w1a = jnp.transpose(W1[:, :D])
    w1b = jnp.transpose(W1[:, D:])
    w2t = jnp.transpose(W2)
    b1r = jnp.broadcast_to(b1[None, :], (8, C1))
    b2r = jnp.broadcast_to(b2[None, :], (8, C2))

    grid = (B, NT)
    y1, stats1 = pl.pallas_call(
        functools.partial(_knn_conv1_kernel, S=S),
        grid=grid,
        in_specs=[
            pl.BlockSpec((1, TN, 3), lambda b, n: (b, n, 0)),
            pl.BlockSpec((1, 3, S), lambda b, n: (b, 0, 0)),
            pl.BlockSpec((1, TN, D), lambda b, n: (b, n, 0)),
            pl.BlockSpec((1, S, D), lambda b, n: (b, 0, 0)),
            pl.BlockSpec((D, C1), lambda b, n: (0, 0)),
            pl.BlockSpec((D, C1), lambda b, n: (0, 0)),
            pl.BlockSpec((8, C1), lambda b, n: (0, 0)),
        ],
        out_specs=[
            pl.BlockSpec((1, TN, C1), lambda b, n: (b, n, 0)),
            pl.BlockSpec((1, 1, 8, C1), lambda b, n: (b, n, 0, 0)),
        ],
        out_shape=[
            jax.ShapeDtypeStruct((B, N, C1), jnp.float32),
            jax.ShapeDtypeStruct((B, NT, 8, C1), jnp.float32),
        ],
    )(xyz1, xyz2t, features1, features2, w1a, w1b, b1r)

    s1 = jnp.sum(stats1, axis=(0, 1))
    mu1 = s1[0] / count
    var1 = s1[1] / count - mu1 * mu1
    alpha1 = g1 * jax.lax.rsqrt(var1 + 1e-5)
    beta1 = be1 - mu1 * alpha1
    ab1 = jnp.zeros((8, C1), jnp.float32)
    ab1 = ab1.at[0].set(alpha1).at[1].set(beta1).at[2, 0].set(a1[0])

    y2, stats2 = pl.pallas_call(
        _bn_conv2_kernel,
        grid=grid,
        in_specs=[
            pl.BlockSpec((1, TN, C1), lambda b, n: (b, n, 0)),
            pl.BlockSpec((8, C1), lambda b, n: (0, 0)),
            pl.BlockSpec((C1, C2), lambda b, n: (0, 0)),
            pl.BlockSpec((8, C2), lambda b, n: (0, 0)),
        ],
        out_specs=[
            pl.BlockSpec((1, TN, C2), lambda b, n: (b, n, 0)),
            pl.BlockSpec((1, 1, 8, C2), lambda b, n: (b, n, 0, 0)),
        ],
        out_shape=[
            jax.ShapeDtypeStruct((B, N, C2), jnp.float32),
            jax.ShapeDtypeStruct((B, NT, 8, C2), jnp.float32),
        ],
    )(y1, ab1, w2t, b2r)

    s2 = jnp.sum(stats2, axis=(0, 1))
    mu2 = s2[0] / count
    var2 = s2[1] / count - mu2 * mu2
    alpha2 = g2 * jax.lax.rsqrt(var2 + 1e-5)
    beta2 = be2 - mu2 * alpha2
    ab2 = jnp.zeros((8, C2), jnp.float32)
    ab2 = ab2.at[0].set(alpha2).at[1].set(beta2).at[2, 0].set(a2[0])

    out = pl.pallas_call(
        _bn_out_kernel,
        grid=grid,
        in_specs=[
            pl.BlockSpec((1, TN, C2), lambda b, n: (b, n, 0)),
            pl.BlockSpec((8, C2), lambda b, n: (0, 0)),
        ],
        out_specs=pl.BlockSpec((1, TN, C2), lambda b, n: (b, n, 0)),
        out_shape=jax.ShapeDtypeStruct((B, N, C2), jnp.float32),
    )(y2, ab2)

    return out


# speedup vs baseline: 32.8528x; 1.3082x over previous
import functools

import jax
import jax.numpy as jnp
from jax.experimental import pallas as pl

_TN = 256


def _knn_conv1_kernel(xyz1_ref, xyz2t_ref, f1_ref, f2_ref, w1a_ref, w1b_ref,
                      b1_ref, y1_ref, stats_ref, *, S):
    x1 = xyz1_ref[0]
    x2t = xyz2t_ref[0]

    dot = jax.lax.dot_general(
        x1.astype(jnp.bfloat16), x2t.astype(jnp.bfloat16),
        dimension_numbers=(((1,), (0,)), ((), ())),
        preferred_element_type=jnp.float32)
    n1 = jnp.sum(x1 * x1, axis=1, keepdims=True)
    n2 = jnp.sum(x2t * x2t, axis=0, keepdims=True)
    d = -2.0 * dot + n1 + n2

    big = jnp.float32(jnp.inf)
    m1 = jnp.min(d, axis=1, keepdims=True)
    dw = jnp.where(d == m1, big, d)
    m2 = jnp.min(dw, axis=1, keepdims=True)
    dw = jnp.where(dw == m2, big, dw)
    m3 = jnp.min(dw, axis=1, keepdims=True)

    r1 = 1.0 / (m1 + 1e-8)
    r2 = 1.0 / (m2 + 1e-8)
    r3 = 1.0 / (m3 + 1e-8)
    norm = r1 + r2 + r3
    zero = jnp.float32(0.0)
    wmat = (jnp.where(d == m1, r1 / norm, zero)
            + jnp.where(d == m2, r2 / norm, zero)
            + jnp.where(d == m3, r3 / norm, zero))

    interp = jnp.dot(wmat, f2_ref[0], preferred_element_type=jnp.float32)

    y = (jnp.dot(f1_ref[0].astype(jnp.bfloat16), w1a_ref[...],
                 preferred_element_type=jnp.float32)
         + jnp.dot(interp.astype(jnp.bfloat16), w1b_ref[...],
                   preferred_element_type=jnp.float32)
         + b1_ref[0:1, :])
    y1_ref[0] = y.astype(jnp.bfloat16)
    stats_ref[0, 0, 0:1, :] = jnp.sum(y, axis=0, keepdims=True)
    stats_ref[0, 0, 1:2, :] = jnp.sum(y * y, axis=0, keepdims=True)


def _bn_conv2_kernel(y1_ref, ab1_ref, w2_ref, b2_ref, y2_ref, stats_ref):
    alpha = ab1_ref[0:1, :]
    beta = ab1_ref[1:2, :]
    a = ab1_ref[2:3, 0:1]
    z = alpha * y1_ref[0].astype(jnp.float32) + beta
    z = jnp.where(z >= 0, z, a * z)
    y = jnp.dot(z.astype(jnp.bfloat16), w2_ref[...],
                preferred_element_type=jnp.float32) + b2_ref[0:1, :]
    y2_ref[0] = y
    stats_ref[0, 0, 0:1, :] = jnp.sum(y, axis=0, keepdims=True)
    stats_ref[0, 0, 1:2, :] = jnp.sum(y * y, axis=0, keepdims=True)


def _bn_out_kernel(y2_ref, ab2_ref, out_ref):
    alpha = ab2_ref[0:1, :]
    beta = ab2_ref[1:2, :]
    a = ab2_ref[2:3, 0:1]
    z = alpha * y2_ref[0] + beta
    out_ref[0] = jnp.where(z >= 0, z, a * z)


def kernel(xyz1, xyz2, features1, features2, W1, b1, g1, be1, a1,
           W2, b2, g2, be2, a2):
    B, N, _ = xyz1.shape
    S = xyz2.shape[1]
    D = features2.shape[-1]
    C1 = W1.shape[0]
    C2 = W2.shape[0]
    TN = _TN
    NT = N // TN
    count = B * N

    xyz2t = jnp.transpose(xyz2, (0, 2, 1))
    w1a = jnp.transpose(W1[:, :D]).astype(jnp.bfloat16)
    w1b = jnp.transpose(W1[:, D:]).astype(jnp.bfloat16)
    w2t = jnp.transpose(W2).astype(jnp.bfloat16)
    b1r = jnp.broadcast_to(b1[None, :], (8, C1))
    b2r = jnp.broadcast_to(b2[None, :], (8, C2))

    grid = (B, NT)
    y1, stats1 = pl.pallas_call(
        functools.partial(_knn_conv1_kernel, S=S),
        grid=grid,
        in_specs=[
            pl.BlockSpec((1, TN, 3), lambda b, n: (b, n, 0)),
            pl.BlockSpec((1, 3, S), lambda b, n: (b, 0, 0)),
            pl.BlockSpec((1, TN, D), lambda b, n: (b, n, 0)),
            pl.BlockSpec((1, S, D), lambda b, n: (b, 0, 0)),
            pl.BlockSpec((D, C1), lambda b, n: (0, 0)),
            pl.BlockSpec((D, C1), lambda b, n: (0, 0)),
            pl.BlockSpec((8, C1), lambda b, n: (0, 0)),
        ],
        out_specs=[
            pl.BlockSpec((1, TN, C1), lambda b, n: (b, n, 0)),
            pl.BlockSpec((1, 1, 8, C1), lambda b, n: (b, n, 0, 0)),
        ],
        out_shape=[
            jax.ShapeDtypeStruct((B, N, C1), jnp.bfloat16),
            jax.ShapeDtypeStruct((B, NT, 8, C1), jnp.float32),
        ],
    )(xyz1, xyz2t, features1, features2, w1a, w1b, b1r)

    s1 = jnp.sum(stats1, axis=(0, 1))
    mu1 = s1[0] / count
    var1 = s1[1] / count - mu1 * mu1
    alpha1 = g1 * jax.lax.rsqrt(var1 + 1e-5)
    beta1 = be1 - mu1 * alpha1
    ab1 = jnp.zeros((8, C1), jnp.float32)
    ab1 = ab1.at[0].set(alpha1).at[1].set(beta1).at[2, 0].set(a1[0])

    y2, stats2 = pl.pallas_call(
        _bn_conv2_kernel,
        grid=grid,
        in_specs=[
            pl.BlockSpec((1, TN, C1), lambda b, n: (b, n, 0)),
            pl.BlockSpec((8, C1), lambda b, n: (0, 0)),
            pl.BlockSpec((C1, C2), lambda b, n: (0, 0)),
            pl.BlockSpec((8, C2), lambda b, n: (0, 0)),
        ],
        out_specs=[
            pl.BlockSpec((1, TN, C2), lambda b, n: (b, n, 0)),
            pl.BlockSpec((1, 1, 8, C2), lambda b, n: (b, n, 0, 0)),
        ],
        out_shape=[
            jax.ShapeDtypeStruct((B, N, C2), jnp.float32),
            jax.ShapeDtypeStruct((B, NT, 8, C2), jnp.float32),
        ],
    )(y1, ab1, w2t, b2r)

    s2 = jnp.sum(stats2, axis=(0, 1))
    mu2 = s2[0] / count
    var2 = s2[1] / count - mu2 * mu2
    alpha2 = g2 * jax.lax.rsqrt(var2 + 1e-5)
    beta2 = be2 - mu2 * alpha2
    ab2 = jnp.zeros((8, C2), jnp.float32)
    ab2 = ab2.at[0].set(alpha2).at[1].set(beta2).at[2, 0].set(a2[0])

    out = pl.pallas_call(
        _bn_out_kernel,
        grid=grid,
        in_specs=[
            pl.BlockSpec((1, TN, C2), lambda b, n: (b, n, 0)),
            pl.BlockSpec((8, C2), lambda b, n: (0, 0)),
        ],
        out_specs=pl.BlockSpec((1, TN, C2), lambda b, n: (b, n, 0)),
        out_shape=jax.ShapeDtypeStruct((B, N, C2), jnp.float32),
    )(y2, ab2)

    return out
